# Initial kernel scaffold; baseline (speedup 1.0000x reference)
import jax
import jax.numpy as jnp
from jax import lax
from jax.experimental import pallas as pl
from jax.experimental.pallas import tpu as pltpu
from jax.experimental.pallas import tpu_sc as plsc

Q = 1024
D = 128
CH = 512
NCH = 196
KPAD = NCH * CH
NKEYS = 100000
MPAD = 256
TOPK = 25
NOUT = 32
ROWS_PER = Q // 32
NEG = -3.0e38
BIG = 2 ** 30


def _k1_body(q_ref, k_ref, sim_ref, m_ref, qn_ref):
    i = pl.program_id(0)

    @pl.when(i == 0)
    def _():
        q = q_ref[...]
        qn = q / jnp.maximum(
            jnp.sqrt(jnp.sum(q * q, axis=1, keepdims=True)), 1e-8)
        qn_ref[...] = qn
        m_ref[...] = jnp.full((Q, MPAD), NEG, jnp.float32)

    kb = k_ref[...]
    kn = kb / jnp.maximum(
        jnp.sqrt(jnp.sum(kb * kb, axis=1, keepdims=True)), 1e-8)
    sim = lax.dot_general(qn_ref[...], kn, (((1,), (1,)), ((), ())),
                          preferred_element_type=jnp.float32,
                          precision=lax.Precision.HIGHEST)
    col = i * CH + lax.broadcasted_iota(jnp.int32, (Q, CH), 1)
    sim = jnp.where(col < NKEYS, sim, NEG)
    sim_ref[...] = sim
    mx = jnp.max(sim, axis=1, keepdims=True)
    lane = lax.broadcasted_iota(jnp.int32, (Q, MPAD), 1)
    m_ref[...] = jnp.where(lane == i, mx, m_ref[...])


def _k1(queries, keys_pad, interpret=False):
    return pl.pallas_call(
        _k1_body,
        grid=(NCH,),
        in_specs=[pl.BlockSpec((Q, D), lambda i: (0, 0)),
                  pl.BlockSpec((CH, D), lambda i: (i, 0))],
        out_specs=[pl.BlockSpec((Q, CH), lambda i: (0, i)),
                   pl.BlockSpec((Q, MPAD), lambda i: (0, 0))],
        out_shape=[jax.ShapeDtypeStruct((Q, KPAD), jnp.float32),
                   jax.ShapeDtypeStruct((Q, MPAD), jnp.float32)],
        scratch_shapes=[pltpu.VMEM((Q, D), jnp.float32)],
        interpret=interpret,
    )(queries, keys_pad)


def _k2_body(simc, mh, vals, idxs, m_buf, cidx, cm, chunks, ovals, oidx, sem):
    c = lax.axis_index("c")
    s = lax.axis_index("s")
    wid = s * 2 + c
    base = wid * ROWS_PER

    lane = lax.iota(jnp.int32, 16)
    onel = lane == 0

    def full_i(v):
        return jnp.full((16,), v, jnp.int32)

    def full_f(v):
        return jnp.full((16,), v, jnp.float32)

    def row_body(rl, _):
        row = base + rl
        pltpu.sync_copy(mh.at[row], m_buf)

        def sel_body(t, _):
            v = m_buf[pl.ds(0, 16)]
            for j in range(1, 16):
                v = jnp.maximum(v, m_buf[pl.ds(j * 16, 16)])
            m = jnp.max(v)
            mi = full_i(BIG)
            for j in range(16):
                w = m_buf[pl.ds(j * 16, 16)]
                mi = jnp.minimum(mi, jnp.where(w == m, lane + j * 16, BIG))
            cix = jnp.min(mi)
            plsc.store_scatter(cidx, [full_i(t)], full_i(cix), mask=onel)
            plsc.store_scatter(cm, [full_i(t)], full_f(m), mask=onel)
            plsc.store_scatter(m_buf, [full_i(cix)], full_f(NEG), mask=onel)
            return 0

        lax.fori_loop(0, TOPK, sel_body, 0)

        cm[pl.ds(16, 16)] = jnp.where(lane + 16 < TOPK, cm[pl.ds(16, 16)],
                                      NEG)
        cidx[pl.ds(16, 16)] = jnp.where(lane + 16 < TOPK,
                                        cidx[pl.ds(16, 16)], 0)

        i0 = cidx[pl.ds(0, 16)] + row * NCH
        i1 = cidx[pl.ds(16, 16)] + row * NCH
        pltpu.async_copy(simc.at[i0], chunks.at[pl.ds(0, 16)], sem).wait()
        pltpu.async_copy(simc.at[i1], chunks.at[pl.ds(16, 16)], sem).wait()

        def ext_body(t, _):
            v0 = cm[pl.ds(0, 16)]
            v1 = cm[pl.ds(16, 16)]
            m = jnp.max(jnp.maximum(v0, v1))
            mi = jnp.where(v0 == m, lane, BIG)
            mi = jnp.minimum(mi, jnp.where(v1 == m, lane + 16, BIG))
            j = jnp.min(mi)
            cg = plsc.load_gather(cidx, [full_i(j)])
            cix = jnp.max(cg)
            pmin = full_i(BIG)
            for w in range(32):
                x = chunks[j, pl.ds(w * 16, 16)]
                pmin = jnp.minimum(pmin,
                                   jnp.where(x == m, lane + w * 16, BIG))
            p = jnp.min(pmin)
            plsc.store_scatter(ovals, [full_i(rl), full_i(t)], full_f(m),
                               mask=onel)
            plsc.store_scatter(oidx, [full_i(rl), full_i(t)],
                               full_i(cix * CH + p), mask=onel)
            plsc.store_scatter(chunks, [full_i(j), full_i(p)], full_f(NEG),
                               mask=onel)
            nv = chunks[j, pl.ds(0, 16)]
            for w in range(1, 32):
                nv = jnp.maximum(nv, chunks[j, pl.ds(w * 16, 16)])
            plsc.store_scatter(cm, [full_i(j)], full_f(jnp.max(nv)),
                               mask=onel)
            return 0

        lax.fori_loop(0, TOPK, ext_body, 0)
        return 0

    lax.fori_loop(0, ROWS_PER, row_body, 0)
    pltpu.sync_copy(ovals, vals.at[pl.ds(base, ROWS_PER)])
    pltpu.sync_copy(oidx, idxs.at[pl.ds(base, ROWS_PER)])


def _k2(simc, mh, interpret=False):
    mesh = plsc.VectorSubcoreMesh(core_axis_name="c", subcore_axis_name="s")
    f = pl.kernel(
        _k2_body,
        out_type=[jax.ShapeDtypeStruct((Q, NOUT), jnp.float32),
                  jax.ShapeDtypeStruct((Q, NOUT), jnp.int32)],
        mesh=mesh,
        scratch_types=[
            pltpu.VMEM((MPAD,), jnp.float32),
            pltpu.VMEM((NOUT,), jnp.int32),
            pltpu.VMEM((NOUT,), jnp.float32),
            pltpu.VMEM((NOUT, CH), jnp.float32),
            pltpu.VMEM((ROWS_PER, NOUT), jnp.float32),
            pltpu.VMEM((ROWS_PER, NOUT), jnp.int32),
            pltpu.SemaphoreType.DMA,
        ],
        interpret=interpret,
    )
    return f(simc, mh)


def kernel(queries, keys, k):
    keys_pad = jnp.pad(keys, ((0, KPAD - keys.shape[0]), (0, 0)))
    sim, mh = _k1(queries, keys_pad)
    simc = sim.reshape(Q * NCH, CH)
    vals, idxs = _k2(simc, mh)
    return vals[:, :TOPK], idxs[:, :TOPK]


# speedup vs baseline: 7.3479x; 7.3479x over previous
import jax
import jax.numpy as jnp
from jax import lax
from jax.experimental import pallas as pl
from jax.experimental.pallas import tpu as pltpu
from jax.experimental.pallas import tpu_sc as plsc

Q = 1024
D = 128
CH = 512
NCH = 196
KPAD = NCH * CH
NKEYS = 100000
MPAD = 256
TOPK = 25
NOUT = 32
ROWS_PER = Q // 32
NEG = -3.0e38
BIG = 2 ** 30


def _k1_body(q_ref, sqq_ref, kt_ref, sqk_ref, sim_ref, m_ref, qn_ref):
    i = pl.program_id(0)

    @pl.when(i == 0)
    def _():
        q = q_ref[...]
        qn = q / jnp.maximum(jnp.sqrt(sqq_ref[...]), 1e-8)
        qn_ref[...] = qn
        m_ref[...] = jnp.full((Q, MPAD), NEG, jnp.float32)

    kt = kt_ref[...]
    nk = jnp.sqrt(sqk_ref[...].reshape(1, CH))
    knt = kt / jnp.maximum(nk, 1e-8)
    sim = lax.dot_general(qn_ref[...], knt, (((1,), (0,)), ((), ())),
                          preferred_element_type=jnp.float32,
                          precision=lax.Precision.DEFAULT)
    col = i * CH + lax.broadcasted_iota(jnp.int32, (Q, CH), 1)
    sim = jnp.where(col < NKEYS, sim, NEG)
    sim_ref[...] = sim
    mx = jnp.max(sim, axis=1, keepdims=True)
    lane = lax.broadcasted_iota(jnp.int32, (Q, MPAD), 1)
    m_ref[...] = jnp.where(lane == i, mx, m_ref[...])


def _k1(queries, sqq, keys_t, sqk, interpret=False):
    return pl.pallas_call(
        _k1_body,
        grid=(NCH,),
        in_specs=[pl.BlockSpec((Q, D), lambda i: (0, 0)),
                  pl.BlockSpec((Q, D), lambda i: (0, 0)),
                  pl.BlockSpec((D, CH), lambda i: (0, i)),
                  pl.BlockSpec((1, 1, CH), lambda i: (i, 0, 0))],
        out_specs=[pl.BlockSpec((Q, CH), lambda i: (0, i)),
                   pl.BlockSpec((Q, MPAD), lambda i: (0, 0))],
        out_shape=[jax.ShapeDtypeStruct((Q, KPAD), jnp.float32),
                   jax.ShapeDtypeStruct((Q, MPAD), jnp.float32)],
        scratch_shapes=[pltpu.VMEM((Q, D), jnp.float32)],
        interpret=interpret,
    )(queries, sqq, keys_t, sqk)


def _k2_body(simc, mh, vals, idxs, m_buf, cidx, cm, chunks, ovals, oidx, sem):
    c = lax.axis_index("c")
    s = lax.axis_index("s")
    wid = s * 2 + c
    base = wid * ROWS_PER

    lane = lax.iota(jnp.int32, 16)
    onel = lane == 0

    def full_i(v):
        return jnp.full((16,), v, jnp.int32)

    def full_f(v):
        return jnp.full((16,), v, jnp.float32)

    def row_body(rl, _):
        row = base + rl
        pltpu.sync_copy(mh.at[row], m_buf)

        def sel_body(t, _):
            v = m_buf[pl.ds(0, 16)]
            for j in range(1, 16):
                v = jnp.maximum(v, m_buf[pl.ds(j * 16, 16)])
            m = jnp.max(v)
            mi = full_i(BIG)
            for j in range(16):
                w = m_buf[pl.ds(j * 16, 16)]
                mi = jnp.minimum(mi, jnp.where(w == m, lane + j * 16, BIG))
            cix = jnp.min(mi)
            plsc.store_scatter(cidx, [full_i(t)], full_i(cix), mask=onel)
            plsc.store_scatter(cm, [full_i(t)], full_f(m), mask=onel)
            plsc.store_scatter(m_buf, [full_i(cix)], full_f(NEG), mask=onel)
            return 0

        lax.fori_loop(0, TOPK, sel_body, 0)

        cm[pl.ds(16, 16)] = jnp.where(lane + 16 < TOPK, cm[pl.ds(16, 16)],
                                      NEG)
        cidx[pl.ds(16, 16)] = jnp.where(lane + 16 < TOPK,
                                        cidx[pl.ds(16, 16)], 0)

        i0 = cidx[pl.ds(0, 16)] + row * NCH
        i1 = cidx[pl.ds(16, 16)] + row * NCH
        pltpu.async_copy(simc.at[i0], chunks.at[pl.ds(0, 16)], sem).wait()
        pltpu.async_copy(simc.at[i1], chunks.at[pl.ds(16, 16)], sem).wait()

        def ext_body(t, _):
            v0 = cm[pl.ds(0, 16)]
            v1 = cm[pl.ds(16, 16)]
            m = jnp.max(jnp.maximum(v0, v1))
            mi = jnp.where(v0 == m, lane, BIG)
            mi = jnp.minimum(mi, jnp.where(v1 == m, lane + 16, BIG))
            j = jnp.min(mi)
            cg = plsc.load_gather(cidx, [full_i(j)])
            cix = jnp.max(cg)
            pmin = full_i(BIG)
            for w in range(32):
                x = chunks[j, pl.ds(w * 16, 16)]
                pmin = jnp.minimum(pmin,
                                   jnp.where(x == m, lane + w * 16, BIG))
            p = jnp.min(pmin)
            plsc.store_scatter(ovals, [full_i(rl), full_i(t)], full_f(m),
                               mask=onel)
            plsc.store_scatter(oidx, [full_i(rl), full_i(t)],
                               full_i(cix * CH + p), mask=onel)
            plsc.store_scatter(chunks, [full_i(j), full_i(p)], full_f(NEG),
                               mask=onel)
            nv = chunks[j, pl.ds(0, 16)]
            for w in range(1, 32):
                nv = jnp.maximum(nv, chunks[j, pl.ds(w * 16, 16)])
            plsc.store_scatter(cm, [full_i(j)], full_f(jnp.max(nv)),
                               mask=onel)
            return 0

        lax.fori_loop(0, TOPK, ext_body, 0)
        return 0

    lax.fori_loop(0, ROWS_PER, row_body, 0)
    pltpu.sync_copy(ovals, vals.at[pl.ds(base, ROWS_PER)])
    pltpu.sync_copy(oidx, idxs.at[pl.ds(base, ROWS_PER)])


def _k2(simc, mh, interpret=False):
    mesh = plsc.VectorSubcoreMesh(core_axis_name="c", subcore_axis_name="s",
                                  num_cores=2, num_subcores=16)
    f = pl.kernel(
        _k2_body,
        out_type=[jax.ShapeDtypeStruct((Q, NOUT), jnp.float32),
                  jax.ShapeDtypeStruct((Q, NOUT), jnp.int32)],
        mesh=mesh,
        scratch_types=[
            pltpu.VMEM((MPAD,), jnp.float32),
            pltpu.VMEM((NOUT,), jnp.int32),
            pltpu.VMEM((NOUT,), jnp.float32),
            pltpu.VMEM((NOUT, CH), jnp.float32),
            pltpu.VMEM((ROWS_PER, NOUT), jnp.float32),
            pltpu.VMEM((ROWS_PER, NOUT), jnp.int32),
            pltpu.SemaphoreType.DMA,
        ],
        compiler_params=pltpu.CompilerParams(needs_layout_passes=False),
        interpret=interpret,
    )
    return f(simc, mh)


def kernel(queries, keys, k):
    keys_pad = jnp.pad(keys, ((0, KPAD - keys.shape[0]), (0, 0)))
    sqq = jnp.broadcast_to(
        jnp.sum(queries * queries, axis=-1, keepdims=True), (Q, D))
    sqk = jnp.pad(jnp.sum(keys * keys, axis=-1),
                  (0, KPAD - keys.shape[0])).reshape(NCH, 1, CH)
    keys_t = keys_pad.T
    sim, mh = _k1(queries, sqq, keys_t, sqk)
    simc = sim.reshape(Q * NCH, CH)
    vals, idxs = _k2(simc, mh)
    return vals[:, :TOPK], idxs[:, :TOPK]


# speedup vs baseline: 7.9129x; 1.0769x over previous
import jax
import jax.numpy as jnp
from jax import lax
from jax.experimental import pallas as pl
from jax.experimental.pallas import tpu as pltpu
from jax.experimental.pallas import tpu_sc as plsc

Q = 1024
D = 128
CH = 512
NCH = 196
KPAD = NCH * CH
NKEYS = 100000
MPAD = 256
TOPK = 25
NOUT = 32
ROWS_PER = Q // 32
NEG = -3.0e38
BIG = 2 ** 30


def _k1_body(q_ref, sqq_ref, kt_ref, sqk_ref, sim_ref, m_ref, qn_ref):
    i = pl.program_id(0)

    @pl.when(i == 0)
    def _():
        q = q_ref[...]
        qn = q / jnp.maximum(jnp.sqrt(sqq_ref[...]), 1e-8)
        qn_ref[...] = qn
        m_ref[...] = jnp.full((Q, MPAD), NEG, jnp.float32)

    kb = kt_ref[...]
    nk = jnp.sqrt(sqk_ref[...].reshape(1, CH))
    nkt = nk.reshape(CH, 1)
    kn = kb / jnp.maximum(nkt, 1e-8)
    sim = lax.dot_general(qn_ref[...], kn, (((1,), (1,)), ((), ())),
                          preferred_element_type=jnp.float32,
                          precision=lax.Precision.DEFAULT)
    col = i * CH + lax.broadcasted_iota(jnp.int32, (Q, CH), 1)
    sim = jnp.where(col < NKEYS, sim, NEG)
    sim_ref[...] = sim
    mx = jnp.max(sim, axis=1, keepdims=True)
    lane = lax.broadcasted_iota(jnp.int32, (Q, MPAD), 1)
    m_ref[...] = jnp.where(lane == i, mx, m_ref[...])


def _k1(queries, sqq, keys_t, sqk, interpret=False):
    return pl.pallas_call(
        _k1_body,
        grid=(NCH,),
        in_specs=[pl.BlockSpec((Q, D), lambda i: (0, 0)),
                  pl.BlockSpec((Q, D), lambda i: (0, 0)),
                  pl.BlockSpec((CH, D), lambda i: (i, 0)),
                  pl.BlockSpec((1, 1, CH), lambda i: (i, 0, 0))],
        out_specs=[pl.BlockSpec((Q, CH), lambda i: (0, i)),
                   pl.BlockSpec((Q, MPAD), lambda i: (0, 0))],
        out_shape=[jax.ShapeDtypeStruct((Q, KPAD), jnp.float32),
                   jax.ShapeDtypeStruct((Q, MPAD), jnp.float32)],
        scratch_shapes=[pltpu.VMEM((Q, D), jnp.float32)],
        interpret=interpret,
    )(queries, sqq, keys_t, sqk)


def _k2_body(simc, mh, vals, idxs, m_buf, cidx, cm, chunks, ovals, oidx, sem):
    c = lax.axis_index("c")
    s = lax.axis_index("s")
    wid = s * 2 + c
    base = wid * ROWS_PER

    lane = lax.iota(jnp.int32, 16)
    onel = lane == 0

    def full_i(v):
        return jnp.full((16,), v, jnp.int32)

    def full_f(v):
        return jnp.full((16,), v, jnp.float32)

    def row_body(rl, _):
        row = base + rl
        pltpu.sync_copy(mh.at[row], m_buf)

        def sel_body(t, _):
            v = m_buf[pl.ds(0, 16)]
            for j in range(1, 16):
                v = jnp.maximum(v, m_buf[pl.ds(j * 16, 16)])
            m = jnp.max(v)
            mi = full_i(BIG)
            for j in range(16):
                w = m_buf[pl.ds(j * 16, 16)]
                mi = jnp.minimum(mi, jnp.where(w == m, lane + j * 16, BIG))
            cix = jnp.min(mi)
            plsc.store_scatter(cidx, [full_i(t)], full_i(cix), mask=onel)
            plsc.store_scatter(cm, [full_i(t)], full_f(m), mask=onel)
            plsc.store_scatter(m_buf, [full_i(cix)], full_f(NEG), mask=onel)
            return 0

        lax.fori_loop(0, TOPK, sel_body, 0)

        cm[pl.ds(16, 16)] = jnp.where(lane + 16 < TOPK, cm[pl.ds(16, 16)],
                                      NEG)
        cidx[pl.ds(16, 16)] = jnp.where(lane + 16 < TOPK,
                                        cidx[pl.ds(16, 16)], 0)

        i0 = cidx[pl.ds(0, 16)] + row * NCH
        i1 = cidx[pl.ds(16, 16)] + row * NCH
        pltpu.async_copy(simc.at[i0], chunks.at[pl.ds(0, 16)], sem).wait()
        pltpu.async_copy(simc.at[i1], chunks.at[pl.ds(16, 16)], sem).wait()

        def ext_body(t, _):
            v0 = cm[pl.ds(0, 16)]
            v1 = cm[pl.ds(16, 16)]
            m = jnp.max(jnp.maximum(v0, v1))
            mi = jnp.where(v0 == m, lane, BIG)
            mi = jnp.minimum(mi, jnp.where(v1 == m, lane + 16, BIG))
            j = jnp.min(mi)
            cg = plsc.load_gather(cidx, [full_i(j)])
            cix = jnp.max(cg)
            pmin = full_i(BIG)
            for w in range(32):
                x = chunks[j, pl.ds(w * 16, 16)]
                pmin = jnp.minimum(pmin,
                                   jnp.where(x == m, lane + w * 16, BIG))
            p = jnp.min(pmin)
            plsc.store_scatter(ovals, [full_i(rl), full_i(t)], full_f(m),
                               mask=onel)
            plsc.store_scatter(oidx, [full_i(rl), full_i(t)],
                               full_i(cix * CH + p), mask=onel)
            plsc.store_scatter(chunks, [full_i(j), full_i(p)], full_f(NEG),
                               mask=onel)
            nv = chunks[j, pl.ds(0, 16)]
            for w in range(1, 32):
                nv = jnp.maximum(nv, chunks[j, pl.ds(w * 16, 16)])
            plsc.store_scatter(cm, [full_i(j)], full_f(jnp.max(nv)),
                               mask=onel)
            return 0

        lax.fori_loop(0, TOPK, ext_body, 0)
        return 0

    lax.fori_loop(0, ROWS_PER, row_body, 0)
    pltpu.sync_copy(ovals, vals.at[pl.ds(base, ROWS_PER)])
    pltpu.sync_copy(oidx, idxs.at[pl.ds(base, ROWS_PER)])


def _k2(simc, mh, interpret=False):
    mesh = plsc.VectorSubcoreMesh(core_axis_name="c", subcore_axis_name="s",
                                  num_cores=2, num_subcores=16)
    f = pl.kernel(
        _k2_body,
        out_type=[jax.ShapeDtypeStruct((Q, NOUT), jnp.float32),
                  jax.ShapeDtypeStruct((Q, NOUT), jnp.int32)],
        mesh=mesh,
        scratch_types=[
            pltpu.VMEM((MPAD,), jnp.float32),
            pltpu.VMEM((NOUT,), jnp.int32),
            pltpu.VMEM((NOUT,), jnp.float32),
            pltpu.VMEM((NOUT, CH), jnp.float32),
            pltpu.VMEM((ROWS_PER, NOUT), jnp.float32),
            pltpu.VMEM((ROWS_PER, NOUT), jnp.int32),
            pltpu.SemaphoreType.DMA,
        ],
        compiler_params=pltpu.CompilerParams(needs_layout_passes=False),
        interpret=interpret,
    )
    return f(simc, mh)


def kernel(queries, keys, k):
    sqq = jnp.broadcast_to(
        jnp.sum(queries * queries, axis=-1, keepdims=True), (Q, D))
    sqk = jnp.pad(jnp.sum(keys * keys, axis=-1),
                  (0, KPAD - keys.shape[0]),
                  constant_values=1.0).reshape(NCH, 1, CH)
    sim, mh = _k1(queries, sqq, keys, sqk)
    simc = sim.reshape(Q * NCH, CH)
    vals, idxs = _k2(simc, mh)
    return vals[:, :TOPK], idxs[:, :TOPK]


# speedup vs baseline: 12.5436x; 1.5852x over previous
import jax
import jax.numpy as jnp
from jax import lax
from jax.experimental import pallas as pl
from jax.experimental.pallas import tpu as pltpu
from jax.experimental.pallas import tpu_sc as plsc

Q = 1024
D = 128
CH = 512
NCH = 196
KPAD = NCH * CH
NKEYS = 100000
MPAD = 256
TOPK = 25
NOUT = 32
ROWS_PER = Q // 32
NEG = -3.0e38
BIG = 2 ** 30


def _k1_body(q_ref, sqq_ref, kt_ref, sqk_ref, sim_ref, m_ref, qn_ref):
    i = pl.program_id(0)

    @pl.when(i == 0)
    def _():
        q = q_ref[...]
        qn = q / jnp.maximum(jnp.sqrt(sqq_ref[...]), 1e-8)
        qn_ref[...] = qn
        m_ref[...] = jnp.full((Q, MPAD), NEG, jnp.float32)

    kb = kt_ref[...]
    nk = jnp.sqrt(sqk_ref[...].reshape(1, CH))
    nkt = nk.reshape(CH, 1)
    kn = kb / jnp.maximum(nkt, 1e-8)
    sim = lax.dot_general(qn_ref[...], kn, (((1,), (1,)), ((), ())),
                          preferred_element_type=jnp.float32,
                          precision=lax.Precision.DEFAULT)
    col = i * CH + lax.broadcasted_iota(jnp.int32, (Q, CH), 1)
    sim = jnp.where(col < NKEYS, sim, NEG)
    sim_ref[...] = sim.reshape(1, Q, CH)
    mx = jnp.max(sim, axis=1, keepdims=True)
    lane = lax.broadcasted_iota(jnp.int32, (Q, MPAD), 1)
    m_ref[...] = jnp.where(lane == i, mx, m_ref[...])


def _k1(queries, sqq, keys_t, sqk, interpret=False):
    return pl.pallas_call(
        _k1_body,
        grid=(NCH,),
        in_specs=[pl.BlockSpec((Q, D), lambda i: (0, 0)),
                  pl.BlockSpec((Q, D), lambda i: (0, 0)),
                  pl.BlockSpec((CH, D), lambda i: (i, 0)),
                  pl.BlockSpec((1, 1, CH), lambda i: (i, 0, 0))],
        out_specs=[pl.BlockSpec((1, Q, CH), lambda i: (i, 0, 0)),
                   pl.BlockSpec((Q, MPAD), lambda i: (0, 0))],
        out_shape=[jax.ShapeDtypeStruct((NCH, Q, CH), jnp.float32),
                   jax.ShapeDtypeStruct((Q, MPAD), jnp.float32)],
        scratch_shapes=[pltpu.VMEM((Q, D), jnp.float32)],
        interpret=interpret,
    )(queries, sqq, keys_t, sqk)


def _k2_body(simc, mh, vals, idxs, m_buf, cidx, cm, chunks, ovals, oidx, sem):
    c = lax.axis_index("c")
    s = lax.axis_index("s")
    wid = s * 2 + c
    base = wid * ROWS_PER

    lane = lax.iota(jnp.int32, 16)
    onel = lane == 0

    def full_i(v):
        return jnp.full((16,), v, jnp.int32)

    def full_f(v):
        return jnp.full((16,), v, jnp.float32)

    def row_body(rl, _):
        row = base + rl
        pltpu.sync_copy(mh.at[row], m_buf)

        def sel_body(t, _):
            v = m_buf[pl.ds(0, 16)]
            for j in range(1, 16):
                v = jnp.maximum(v, m_buf[pl.ds(j * 16, 16)])
            m = jnp.max(v)
            mi = full_i(BIG)
            for j in range(16):
                w = m_buf[pl.ds(j * 16, 16)]
                mi = jnp.minimum(mi, jnp.where(w == m, lane + j * 16, BIG))
            cix = jnp.min(mi)
            plsc.store_scatter(cidx, [full_i(t)], full_i(cix), mask=onel)
            plsc.store_scatter(cm, [full_i(t)], full_f(m), mask=onel)
            plsc.store_scatter(m_buf, [full_i(cix)], full_f(NEG), mask=onel)
            return 0

        lax.fori_loop(0, TOPK, sel_body, 0)

        cm[pl.ds(16, 16)] = jnp.where(lane + 16 < TOPK, cm[pl.ds(16, 16)],
                                      NEG)
        cidx[pl.ds(16, 16)] = jnp.where(lane + 16 < TOPK,
                                        cidx[pl.ds(16, 16)], 0)

        i0 = cidx[pl.ds(0, 16)] * Q + row
        i1 = cidx[pl.ds(16, 16)] * Q + row
        pltpu.async_copy(simc.at[i0], chunks.at[pl.ds(0, 16)], sem).wait()
        pltpu.async_copy(simc.at[i1], chunks.at[pl.ds(16, 16)], sem).wait()

        def ext_body(t, _):
            v0 = cm[pl.ds(0, 16)]
            v1 = cm[pl.ds(16, 16)]
            m = jnp.max(jnp.maximum(v0, v1))
            mi = jnp.where(v0 == m, lane, BIG)
            mi = jnp.minimum(mi, jnp.where(v1 == m, lane + 16, BIG))
            j = jnp.min(mi)
            cg = plsc.load_gather(cidx, [full_i(j)])
            cix = jnp.max(cg)
            pmin = full_i(BIG)
            for w in range(32):
                x = chunks[j, pl.ds(w * 16, 16)]
                pmin = jnp.minimum(pmin,
                                   jnp.where(x == m, lane + w * 16, BIG))
            p = jnp.min(pmin)
            plsc.store_scatter(ovals, [full_i(rl), full_i(t)], full_f(m),
                               mask=onel)
            plsc.store_scatter(oidx, [full_i(rl), full_i(t)],
                               full_i(cix * CH + p), mask=onel)
            plsc.store_scatter(chunks, [full_i(j), full_i(p)], full_f(NEG),
                               mask=onel)
            nv = chunks[j, pl.ds(0, 16)]
            for w in range(1, 32):
                nv = jnp.maximum(nv, chunks[j, pl.ds(w * 16, 16)])
            plsc.store_scatter(cm, [full_i(j)], full_f(jnp.max(nv)),
                               mask=onel)
            return 0

        lax.fori_loop(0, TOPK, ext_body, 0)
        return 0

    lax.fori_loop(0, ROWS_PER, row_body, 0)
    pltpu.sync_copy(ovals, vals.at[pl.ds(base, ROWS_PER)])
    pltpu.sync_copy(oidx, idxs.at[pl.ds(base, ROWS_PER)])


def _k2(simc, mh, interpret=False):
    mesh = plsc.VectorSubcoreMesh(core_axis_name="c", subcore_axis_name="s",
                                  num_cores=2, num_subcores=16)
    f = pl.kernel(
        _k2_body,
        out_type=[jax.ShapeDtypeStruct((Q, NOUT), jnp.float32),
                  jax.ShapeDtypeStruct((Q, NOUT), jnp.int32)],
        mesh=mesh,
        scratch_types=[
            pltpu.VMEM((MPAD,), jnp.float32),
            pltpu.VMEM((NOUT,), jnp.int32),
            pltpu.VMEM((NOUT,), jnp.float32),
            pltpu.VMEM((NOUT, CH), jnp.float32),
            pltpu.VMEM((ROWS_PER, NOUT), jnp.float32),
            pltpu.VMEM((ROWS_PER, NOUT), jnp.int32),
            pltpu.SemaphoreType.DMA,
        ],
        compiler_params=pltpu.CompilerParams(needs_layout_passes=False),
        interpret=interpret,
    )
    return f(simc, mh)


def kernel(queries, keys, k):
    sqq = jnp.broadcast_to(
        jnp.sum(queries * queries, axis=-1, keepdims=True), (Q, D))
    sqk = jnp.pad(jnp.sum(keys * keys, axis=-1),
                  (0, KPAD - keys.shape[0]),
                  constant_values=1.0).reshape(NCH, 1, CH)
    sim, mh = _k1(queries, sqq, keys, sqk)
    simc = sim.reshape(NCH * Q, CH)
    vals, idxs = _k2(simc, mh)
    return vals[:, :TOPK], idxs[:, :TOPK]


# speedup vs baseline: 14.3145x; 1.1412x over previous
import jax
import jax.numpy as jnp
from jax import lax
from jax.experimental import pallas as pl
from jax.experimental.pallas import tpu as pltpu
from jax.experimental.pallas import tpu_sc as plsc

Q = 1024
D = 128
CH = 512
NCH = 196
KPAD = NCH * CH
NKEYS = 100000
MPAD = 256
TOPK = 25
NOUT = 32
ROWS_PER = Q // 32
NEG = -3.0e38
BIG = 2 ** 30


def _k1_body(q_ref, sqq_ref, kt_ref, sqk_ref, sim_ref, m_ref, qn_ref):
    i = pl.program_id(0)

    @pl.when(i == 0)
    def _():
        q = q_ref[...]
        qn = q / jnp.maximum(jnp.sqrt(sqq_ref[...]), 1e-8)
        qn_ref[...] = qn
        m_ref[...] = jnp.full((Q, MPAD), NEG, jnp.float32)

    kb = kt_ref[...]
    nk = jnp.sqrt(sqk_ref[...].reshape(1, CH))
    nkt = nk.reshape(CH, 1)
    kn = kb / jnp.maximum(nkt, 1e-8)
    sim = lax.dot_general(qn_ref[...], kn, (((1,), (1,)), ((), ())),
                          preferred_element_type=jnp.float32,
                          precision=lax.Precision.DEFAULT)
    col = i * CH + lax.broadcasted_iota(jnp.int32, (Q, CH), 1)
    sim = jnp.where(col < NKEYS, sim, NEG)
    sim_ref[...] = sim.reshape(1, Q, CH)
    mx = jnp.max(sim, axis=1, keepdims=True)
    lane = lax.broadcasted_iota(jnp.int32, (Q, MPAD), 1)
    m_ref[...] = jnp.where(lane == i, mx, m_ref[...])


def _k1(queries, sqq, keys_t, sqk, interpret=False):
    return pl.pallas_call(
        _k1_body,
        grid=(NCH,),
        in_specs=[pl.BlockSpec((Q, D), lambda i: (0, 0)),
                  pl.BlockSpec((Q, D), lambda i: (0, 0)),
                  pl.BlockSpec((CH, D), lambda i: (i, 0)),
                  pl.BlockSpec((1, 1, CH), lambda i: (i, 0, 0))],
        out_specs=[pl.BlockSpec((1, Q, CH), lambda i: (i, 0, 0)),
                   pl.BlockSpec((Q, MPAD), lambda i: (0, 0))],
        out_shape=[jax.ShapeDtypeStruct((NCH, Q, CH), jnp.float32),
                   jax.ShapeDtypeStruct((Q, MPAD), jnp.float32)],
        scratch_shapes=[pltpu.VMEM((Q, D), jnp.float32)],
        interpret=interpret,
    )(queries, sqq, keys_t, sqk)


def _k2_body(simc, mh, vals, idxs, m_buf, cidx, cm, chunks, ovals, oidx,
             sem, msem):
    c = lax.axis_index("c")
    s = lax.axis_index("s")
    wid = s * 2 + c
    base = wid * ROWS_PER

    lane = lax.iota(jnp.int32, 16)
    onel = lane == 0

    def full_i(v):
        return jnp.full((16,), v, jnp.int32)

    def full_f(v):
        return jnp.full((16,), v, jnp.float32)

    pltpu.make_async_copy(mh.at[base], m_buf.at[0], msem).start()

    def step(r, _):
        rb = lax.rem(r, 2)
        pb = lax.rem(r + 1, 2)

        @pl.when(r < ROWS_PER)
        def _():
            row = base + r
            pltpu.make_async_copy(mh.at[row], m_buf.at[rb], msem).wait()

            @pl.when(r + 1 < ROWS_PER)
            def _():
                pltpu.make_async_copy(mh.at[row + 1], m_buf.at[pb],
                                      msem).start()

            def sel_body(t, _):
                vmax = m_buf[rb, pl.ds(0, 16)]
                vidx = lane
                for w in range(1, 16):
                    x = m_buf[rb, pl.ds(w * 16, 16)]
                    gt = x > vmax
                    vmax = jnp.where(gt, x, vmax)
                    vidx = jnp.where(gt, lane + w * 16, vidx)
                m = jnp.max(vmax)
                cix = jnp.min(jnp.where(vmax == m, vidx, BIG))
                plsc.store_scatter(cidx, [full_i(rb), full_i(t)],
                                   full_i(cix), mask=onel)
                plsc.store_scatter(cm, [full_i(rb), full_i(t)],
                                   full_f(m), mask=onel)
                plsc.store_scatter(m_buf, [full_i(rb), full_i(cix)],
                                   full_f(NEG), mask=onel)
                return 0

            lax.fori_loop(0, TOPK, sel_body, 0)

            cm[rb, pl.ds(16, 16)] = jnp.where(
                lane + 16 < TOPK, cm[rb, pl.ds(16, 16)], NEG)
            cidx[rb, pl.ds(16, 16)] = jnp.where(
                lane + 16 < TOPK, cidx[rb, pl.ds(16, 16)], 0)

        @pl.when(r > 0)
        def _():
            pltpu.make_async_copy(simc.at[pl.ds(0, NOUT)],
                                  chunks.at[lax.rem(r - 1, 2)], sem).wait()

        @pl.when(r < ROWS_PER)
        def _():
            row = base + r
            i0 = cidx[rb, pl.ds(0, 16)] * Q + row
            i1 = cidx[rb, pl.ds(16, 16)] * Q + row
            pltpu.async_copy(simc.at[i0], chunks.at[rb, pl.ds(0, 16)], sem)
            pltpu.async_copy(simc.at[i1], chunks.at[rb, pl.ds(16, 16)], sem)

        @pl.when(r > 0)
        def _():
            rl = r - 1
            qb = lax.rem(rl, 2)

            def ext_body(t, _):
                v0 = cm[qb, pl.ds(0, 16)]
                v1 = cm[qb, pl.ds(16, 16)]
                m = jnp.max(jnp.maximum(v0, v1))
                mi = jnp.where(v0 == m, lane, BIG)
                mi = jnp.minimum(mi, jnp.where(v1 == m, lane + 16, BIG))
                j = jnp.min(mi)
                cg = plsc.load_gather(cidx, [full_i(qb), full_i(j)])
                cix = jnp.max(cg)
                pmin = full_i(BIG)
                ieq = jnp.zeros((16,), jnp.int32)
                nmaxv = full_f(NEG)
                for w in range(32):
                    x = chunks[qb, j, pl.ds(w * 16, 16)]
                    eq = x == m
                    pmin = jnp.minimum(pmin,
                                       jnp.where(eq, lane + w * 16, BIG))
                    ieq = ieq + jnp.where(eq, 1, 0)
                    nmaxv = jnp.maximum(nmaxv, jnp.where(eq, NEG, x))
                p = jnp.min(pmin)
                neq = jnp.sum(ieq)
                ncm = jnp.where(neq >= 2, m, jnp.max(nmaxv))
                plsc.store_scatter(ovals, [full_i(rl), full_i(t)],
                                   full_f(m), mask=onel)
                plsc.store_scatter(oidx, [full_i(rl), full_i(t)],
                                   full_i(cix * CH + p), mask=onel)
                plsc.store_scatter(
                    chunks, [full_i(qb), full_i(j), full_i(p)],
                    full_f(NEG), mask=onel)
                plsc.store_scatter(cm, [full_i(qb), full_i(j)],
                                   full_f(ncm), mask=onel)
                return 0

            lax.fori_loop(0, TOPK, ext_body, 0)

        return 0

    lax.fori_loop(0, ROWS_PER + 1, step, 0)
    pltpu.sync_copy(ovals, vals.at[pl.ds(base, ROWS_PER)])
    pltpu.sync_copy(oidx, idxs.at[pl.ds(base, ROWS_PER)])


def _k2(simc, mh, interpret=False):
    mesh = plsc.VectorSubcoreMesh(core_axis_name="c", subcore_axis_name="s",
                                  num_cores=2, num_subcores=16)
    f = pl.kernel(
        _k2_body,
        out_type=[jax.ShapeDtypeStruct((Q, NOUT), jnp.float32),
                  jax.ShapeDtypeStruct((Q, NOUT), jnp.int32)],
        mesh=mesh,
        scratch_types=[
            pltpu.VMEM((2, MPAD), jnp.float32),
            pltpu.VMEM((2, NOUT), jnp.int32),
            pltpu.VMEM((2, NOUT), jnp.float32),
            pltpu.VMEM((2, NOUT, CH), jnp.float32),
            pltpu.VMEM((ROWS_PER, NOUT), jnp.float32),
            pltpu.VMEM((ROWS_PER, NOUT), jnp.int32),
            pltpu.SemaphoreType.DMA,
            pltpu.SemaphoreType.DMA,
        ],
        compiler_params=pltpu.CompilerParams(needs_layout_passes=False),
        interpret=interpret,
    )
    return f(simc, mh)


def kernel(queries, keys, k):
    sqq = jnp.broadcast_to(
        jnp.sum(queries * queries, axis=-1, keepdims=True), (Q, D))
    sqk = jnp.pad(jnp.sum(keys * keys, axis=-1),
                  (0, KPAD - keys.shape[0]),
                  constant_values=1.0).reshape(NCH, 1, CH)
    sim, mh = _k1(queries, sqq, keys, sqk)
    simc = sim.reshape(NCH * Q, CH)
    vals, idxs = _k2(simc, mh)
    return vals[:, :TOPK], idxs[:, :TOPK]
